# baseline (speedup 1.0000x reference)
import jax
import jax.numpy as jnp
from jax import lax
from jax.experimental import pallas as pl
from jax.experimental.pallas import tpu as pltpu
from jax.experimental.pallas import tpu_sc as plsc

_N = 6_400_000
_NUM_SEG = 10_000
_SEG_PAD = 10_240
_NC, _NS = 2, 16
_NW = _NC * _NS
_NODES_W = _N // _NW
_P = 4_000
_NPIECES = _NODES_W // _P
_GROUPS = _P // 16
_SEG_W = _SEG_PAD // _NS


def _body(pos_hbm, ids_hbm, min_hbm, eng_out, grad_out,
          pos_buf, ids_buf, grad_buf, eng_buf, red_buf, min_buf, shared):
  cid = lax.axis_index("c")
  sid = lax.axis_index("s")
  wid = sid * _NC + cid
  node0 = wid * _NODES_W

  pltpu.sync_copy(min_hbm, min_buf)
  minx = min_buf[pl.ds(0, 16)]
  miny = min_buf[pl.ds(16, 16)]
  minz = min_buf[pl.ds(32, 16)]

  zeros16 = jnp.zeros((16,), jnp.float32)

  def zero_body(j, _):
    eng_buf[pl.ds(j * 16, 16)] = zeros16
    return 0
  lax.fori_loop(0, _SEG_PAD // 16, zero_body, 0)

  iota = lax.iota(jnp.int32, 16)
  t3 = iota * 3
  up = jnp.minimum(iota + 1, 15)
  dn = jnp.maximum(iota - 1, 0)
  is0 = iota == 0
  is15 = iota == 15

  def piece_body(k, _):
    start = node0 + k * _P
    pltpu.sync_copy(pos_hbm.at[pl.ds(start * 3, 3 * _P)], pos_buf)
    pltpu.sync_copy(ids_hbm.at[pl.ds(start, _P)], ids_buf)

    def group_body(g, _):
      ids16 = ids_buf[pl.ds(g * 16, 16)]
      ix = jnp.full((16,), g * 48, jnp.int32) + t3
      iy = ix + 1
      iz = ix + 2
      x = plsc.load_gather(pos_buf, [ix])
      y = plsc.load_gather(pos_buf, [iy])
      z = plsc.load_gather(pos_buf, [iz])
      dx = x - minx
      dy = y - miny
      dz = z - minz
      sq = dx * dx + dy * dy + dz * dz
      c = plsc.cumsum(sq)
      ids_up = jnp.take_along_axis(ids16, up, axis=0, mode="promise_in_bounds")
      ids_dn = jnp.take_along_axis(ids16, dn, axis=0, mode="promise_in_bounds")
      c_dn = jnp.take_along_axis(c, dn, axis=0, mode="promise_in_bounds")
      m_end = (ids_up != ids16) | is15
      m_start = (ids16 != ids_dn) | is0
      neg_prev = jnp.where(is0, 0.0, -c_dn)
      plsc.addupdate_scatter(eng_buf, [ids16], c, mask=m_end)
      plsc.addupdate_scatter(eng_buf, [ids16], neg_prev, mask=m_start)
      plsc.store_scatter(grad_buf, [ix], -2.0 * dx)
      plsc.store_scatter(grad_buf, [iy], -2.0 * dy)
      plsc.store_scatter(grad_buf, [iz], -2.0 * dz)
      return 0

    lax.fori_loop(0, _GROUPS, group_body, 0)
    pltpu.sync_copy(grad_buf, grad_out.at[pl.ds(start * 3, 3 * _P)])
    return 0

  lax.fori_loop(0, _NPIECES, piece_body, 0)

  pltpu.sync_copy(eng_buf, shared.at[sid])
  plsc.subcore_barrier()
  col0 = sid * _SEG_W
  for r in range(_NS):
    pltpu.sync_copy(shared.at[r, pl.ds(col0, _SEG_W)], red_buf.at[r])

  def red_body(j, _):
    acc = red_buf[0, pl.ds(j * 16, 16)]
    for r in range(1, _NS):
      acc = acc + red_buf[r, pl.ds(j * 16, 16)]
    red_buf[0, pl.ds(j * 16, 16)] = acc
    return 0
  lax.fori_loop(0, _SEG_W // 16, red_body, 0)
  pltpu.sync_copy(red_buf.at[0], eng_out.at[cid, pl.ds(col0, _SEG_W)])


_sc_call_cache = []


def _sc_call(*args):
  if not _sc_call_cache:
    _sc_call_cache.append(_make_sc_call())
  return _sc_call_cache[0](*args)


def _make_sc_call():
  return pl.kernel(
    _body,
    out_type=(
        jax.ShapeDtypeStruct((_NC, _SEG_PAD), jnp.float32),
        jax.ShapeDtypeStruct((3 * _N,), jnp.float32),
    ),
    mesh=plsc.VectorSubcoreMesh(core_axis_name="c", subcore_axis_name="s",
                                num_cores=_NC, num_subcores=_NS),
    scratch_types=[
        pltpu.VMEM((3 * _P,), jnp.float32),
        pltpu.VMEM((_P,), jnp.int32),
        pltpu.VMEM((3 * _P,), jnp.float32),
        pltpu.VMEM((_SEG_PAD,), jnp.float32),
        pltpu.VMEM((_NS, _SEG_W), jnp.float32),
        pltpu.VMEM((48,), jnp.float32),
        pltpu.VMEM_SHARED((_NS, _SEG_PAD), jnp.float32),
    ],
    compiler_params=pltpu.CompilerParams(needs_layout_passes=False),
  )


def kernel(positions, segment_ids, minimum):
  pos_flat = positions.reshape(-1)
  min_bcast = jnp.repeat(minimum.astype(jnp.float32), 16)
  eng2, grad_flat = _sc_call(pos_flat, segment_ids, min_bcast)
  energies = (eng2[0] + eng2[1])[:_NUM_SEG]
  neg_grad = grad_flat.reshape(_N, 3)
  stress = jnp.zeros((_NUM_SEG, 6), positions.dtype)
  return (energies, neg_grad, stress)


# speedup vs baseline: 20.3267x; 20.3267x over previous
import jax
import jax.numpy as jnp
from jax import lax
from jax.experimental import pallas as pl
from jax.experimental.pallas import tpu as pltpu
from jax.experimental.pallas import tpu_sc as plsc

_N = 6_400_000
_NUM_SEG = 10_000
_SEG_PAD = 10_240
_NC, _NS = 2, 16
_NW = _NC * _NS
_NODES_W = _N // _NW
_P = 4_000
_NPIECES = _NODES_W // _P
_GROUPS = _P // 16
_SEG_W = _SEG_PAD // _NS


def _body(xs_hbm, ys_hbm, zs_hbm, ids_hbm, min_hbm,
          eng_out, gx_out, gy_out, gz_out,
          xb, yb, zb, ids_buf, gxb, gyb, gzb,
          eng_buf, red_buf, min_buf, shared):
  cid = lax.axis_index("c")
  sid = lax.axis_index("s")
  wid = sid * _NC + cid
  node0 = wid * _NODES_W

  pltpu.sync_copy(min_hbm, min_buf)
  minx = min_buf[pl.ds(0, 16)]
  miny = min_buf[pl.ds(16, 16)]
  minz = min_buf[pl.ds(32, 16)]

  zeros16 = jnp.zeros((16,), jnp.float32)

  def zero_body(j, _):
    eng_buf[pl.ds(j * 16, 16)] = zeros16
    return 0
  lax.fori_loop(0, _SEG_PAD // 16, zero_body, 0)

  iota = lax.iota(jnp.int32, 16)
  up = jnp.minimum(iota + 1, 15)
  dn = jnp.maximum(iota - 1, 0)
  is0 = iota == 0
  is15 = iota == 15

  def piece_body(k, _):
    start = node0 + k * _P
    pltpu.sync_copy(xs_hbm.at[pl.ds(start, _P)], xb)
    pltpu.sync_copy(ys_hbm.at[pl.ds(start, _P)], yb)
    pltpu.sync_copy(zs_hbm.at[pl.ds(start, _P)], zb)
    pltpu.sync_copy(ids_hbm.at[pl.ds(start, _P)], ids_buf)

    def group_body(g, _):
      o = pl.ds(g * 16, 16)
      ids16 = ids_buf[o]
      dx = xb[o] - minx
      dy = yb[o] - miny
      dz = zb[o] - minz
      sq = dx * dx + dy * dy + dz * dz
      c = plsc.cumsum(sq)
      ids_up = jnp.take_along_axis(ids16, up, axis=0, mode="promise_in_bounds")
      ids_dn = jnp.take_along_axis(ids16, dn, axis=0, mode="promise_in_bounds")
      c_dn = jnp.take_along_axis(c, dn, axis=0, mode="promise_in_bounds")
      m_end = (ids_up != ids16) | is15
      m_start = (ids16 != ids_dn) | is0
      neg_prev = jnp.where(is0, 0.0, -c_dn)
      plsc.addupdate_scatter(eng_buf, [ids16], c, mask=m_end)
      plsc.addupdate_scatter(eng_buf, [ids16], neg_prev, mask=m_start)
      gxb[o] = -2.0 * dx
      gyb[o] = -2.0 * dy
      gzb[o] = -2.0 * dz
      return 0

    lax.fori_loop(0, _GROUPS, group_body, 0)
    pltpu.sync_copy(gxb, gx_out.at[pl.ds(start, _P)])
    pltpu.sync_copy(gyb, gy_out.at[pl.ds(start, _P)])
    pltpu.sync_copy(gzb, gz_out.at[pl.ds(start, _P)])
    return 0

  lax.fori_loop(0, _NPIECES, piece_body, 0)

  pltpu.sync_copy(eng_buf, shared.at[sid])
  plsc.subcore_barrier()
  seg0 = sid * _SEG_W
  for r in range(_NS):
    pltpu.sync_copy(shared.at[r, pl.ds(seg0, _SEG_W)], red_buf.at[r])

  def red_body(j, _):
    acc = red_buf[0, pl.ds(j * 16, 16)]
    for r in range(1, _NS):
      acc = acc + red_buf[r, pl.ds(j * 16, 16)]
    red_buf[0, pl.ds(j * 16, 16)] = acc
    return 0
  lax.fori_loop(0, _SEG_W // 16, red_body, 0)
  pltpu.sync_copy(red_buf.at[0], eng_out.at[cid, pl.ds(seg0, _SEG_W)])


_sc_call_cache = []


def _sc_call(*args):
  if not _sc_call_cache:
    _sc_call_cache.append(_make_sc_call())
  return _sc_call_cache[0](*args)


def _make_sc_call():
  return pl.kernel(
    _body,
    out_type=(
        jax.ShapeDtypeStruct((_NC, _SEG_PAD), jnp.float32),
        jax.ShapeDtypeStruct((_N,), jnp.float32),
        jax.ShapeDtypeStruct((_N,), jnp.float32),
        jax.ShapeDtypeStruct((_N,), jnp.float32),
    ),
    mesh=plsc.VectorSubcoreMesh(core_axis_name="c", subcore_axis_name="s",
                                num_cores=_NC, num_subcores=_NS),
    scratch_types=[
        pltpu.VMEM((_P,), jnp.float32),
        pltpu.VMEM((_P,), jnp.float32),
        pltpu.VMEM((_P,), jnp.float32),
        pltpu.VMEM((_P,), jnp.int32),
        pltpu.VMEM((_P,), jnp.float32),
        pltpu.VMEM((_P,), jnp.float32),
        pltpu.VMEM((_P,), jnp.float32),
        pltpu.VMEM((_SEG_PAD,), jnp.float32),
        pltpu.VMEM((_NS, _SEG_W), jnp.float32),
        pltpu.VMEM((48,), jnp.float32),
        pltpu.VMEM_SHARED((_NS, _SEG_PAD), jnp.float32),
    ],
    compiler_params=pltpu.CompilerParams(needs_layout_passes=False),
  )


def kernel(positions, segment_ids, minimum):
  xs = positions[:, 0]
  ys = positions[:, 1]
  zs = positions[:, 2]
  min_bcast = jnp.repeat(minimum.astype(jnp.float32), 16)
  eng2, gx, gy, gz = _sc_call(xs, ys, zs, segment_ids, min_bcast)
  energies = (eng2[0] + eng2[1])[:_NUM_SEG]
  neg_grad = jnp.stack([gx, gy, gz], axis=1)
  stress = jnp.zeros((_NUM_SEG, 6), positions.dtype)
  return (energies, neg_grad, stress)


# speedup vs baseline: 42.9684x; 2.1139x over previous
import jax
import jax.numpy as jnp
from jax import lax
from jax.experimental import pallas as pl
from jax.experimental.pallas import tpu as pltpu
from jax.experimental.pallas import tpu_sc as plsc

_N = 6_400_000
_NUM_SEG = 10_000
_SEG_PAD = 10_240
_NC, _NS = 2, 16
_NW = _NC * _NS
_NODES_W = _N // _NW
_P = 4_000
_NPIECES = _NODES_W // _P
_GROUPS = _P // 16
_SEG_W = _SEG_PAD // _NS


def _body(xs_hbm, ys_hbm, zs_hbm, ids_hbm, min_hbm,
          eng_out, gx_out, gy_out, gz_out,
          xb0, xb1, yb0, yb1, zb0, zb1, ib0, ib1,
          gxb0, gxb1, gyb0, gyb1, gzb0, gzb1,
          eng_buf, red_buf, min_buf, shared,
          in_sem0, in_sem1, out_sem0, out_sem1):
  cid = lax.axis_index("c")
  sid = lax.axis_index("s")
  wid = sid * _NC + cid
  node0 = wid * _NODES_W

  pltpu.sync_copy(min_hbm, min_buf)
  minx = min_buf[pl.ds(0, 16)]
  miny = min_buf[pl.ds(16, 16)]
  minz = min_buf[pl.ds(32, 16)]

  zeros16 = jnp.zeros((16,), jnp.float32)

  def zero_body(j, _):
    eng_buf[pl.ds(j * 16, 16)] = zeros16
    return 0
  lax.fori_loop(0, _SEG_PAD // 16, zero_body, 0)

  iota = lax.iota(jnp.int32, 16)
  up = jnp.minimum(iota + 1, 15)
  dn = jnp.maximum(iota - 1, 0)
  is0 = iota == 0
  is15 = iota == 15

  in_bufs = (((xb0, xs_hbm), (yb0, ys_hbm), (zb0, zs_hbm), (ib0, ids_hbm)),
             ((xb1, xs_hbm), (yb1, ys_hbm), (zb1, zs_hbm), (ib1, ids_hbm)))
  out_bufs = (((gxb0, gx_out), (gyb0, gy_out), (gzb0, gz_out)),
              ((gxb1, gx_out), (gyb1, gy_out), (gzb1, gz_out)))
  in_sems = (in_sem0, in_sem1)
  out_sems = (out_sem0, out_sem1)

  def start_in(p, b):
    src = pl.ds(node0 + p * _P, _P)
    for buf, hbm in in_bufs[b]:
      pltpu.async_copy(hbm.at[src], buf, in_sems[b])

  def wait_in(p, b):
    src = pl.ds(node0 + p * _P, _P)
    for buf, hbm in in_bufs[b]:
      pltpu.make_async_copy(hbm.at[src], buf, in_sems[b]).wait()

  def start_out(p, b):
    dst = pl.ds(node0 + p * _P, _P)
    for buf, hbm in out_bufs[b]:
      pltpu.async_copy(buf, hbm.at[dst], out_sems[b])

  def wait_out(p, b):
    dst = pl.ds(node0 + p * _P, _P)
    for buf, hbm in out_bufs[b]:
      pltpu.make_async_copy(buf, hbm.at[dst], out_sems[b]).wait()

  def compute_piece(b):
    (xb, _), (yb, _), (zb, _), (ib, _) = in_bufs[b]
    (gxb, _), (gyb, _), (gzb, _) = out_bufs[b]

    def group_body(g):
      o = pl.ds(g * 16, 16)
      ids16 = ib[o]
      dx = xb[o] - minx
      dy = yb[o] - miny
      dz = zb[o] - minz
      sq = dx * dx + dy * dy + dz * dz
      c = plsc.cumsum(sq)
      ids_up = jnp.take_along_axis(ids16, up, axis=0, mode="promise_in_bounds")
      ids_dn = jnp.take_along_axis(ids16, dn, axis=0, mode="promise_in_bounds")
      c_dn = jnp.take_along_axis(c, dn, axis=0, mode="promise_in_bounds")
      m_end = (ids_up != ids16) | is15
      m_start = (ids16 != ids_dn) | is0
      neg_prev = jnp.where(is0, 0.0, -c_dn)
      plsc.addupdate_scatter(eng_buf, [ids16], c, mask=m_end)
      plsc.addupdate_scatter(eng_buf, [ids16], neg_prev, mask=m_start)
      gxb[o] = -2.0 * dx
      gyb[o] = -2.0 * dy
      gzb[o] = -2.0 * dz

    plsc.parallel_loop(0, _GROUPS, 1, unroll=4)(group_body)

  start_in(0, 0)
  start_in(1, 1)

  def pair_body(k2, _):
    for b in range(2):
      p = k2 * 2 + b

      @pl.when(p >= 2)
      def _():
        wait_out(p - 2, b)
      wait_in(p, b)
      compute_piece(b)
      start_out(p, b)

      @pl.when(p + 2 < _NPIECES)
      def _():
        start_in(p + 2, b)
    return 0

  lax.fori_loop(0, _NPIECES // 2, pair_body, 0)
  wait_out(_NPIECES - 2, 0)
  wait_out(_NPIECES - 1, 1)

  pltpu.sync_copy(eng_buf, shared.at[sid])
  plsc.subcore_barrier()
  seg0 = sid * _SEG_W
  for r in range(_NS):
    pltpu.sync_copy(shared.at[r, pl.ds(seg0, _SEG_W)], red_buf.at[r])

  def red_body(j, _):
    acc = red_buf[0, pl.ds(j * 16, 16)]
    for r in range(1, _NS):
      acc = acc + red_buf[r, pl.ds(j * 16, 16)]
    red_buf[0, pl.ds(j * 16, 16)] = acc
    return 0
  lax.fori_loop(0, _SEG_W // 16, red_body, 0)
  pltpu.sync_copy(red_buf.at[0], eng_out.at[cid, pl.ds(seg0, _SEG_W)])


_sc_call_cache = []


def _sc_call(*args):
  if not _sc_call_cache:
    _sc_call_cache.append(_make_sc_call())
  return _sc_call_cache[0](*args)


def _make_sc_call():
  return pl.kernel(
    _body,
    out_type=(
        jax.ShapeDtypeStruct((_NC, _SEG_PAD), jnp.float32),
        jax.ShapeDtypeStruct((_N,), jnp.float32),
        jax.ShapeDtypeStruct((_N,), jnp.float32),
        jax.ShapeDtypeStruct((_N,), jnp.float32),
    ),
    mesh=plsc.VectorSubcoreMesh(core_axis_name="c", subcore_axis_name="s",
                                num_cores=_NC, num_subcores=_NS),
    scratch_types=[
        pltpu.VMEM((_P,), jnp.float32),
        pltpu.VMEM((_P,), jnp.float32),
        pltpu.VMEM((_P,), jnp.float32),
        pltpu.VMEM((_P,), jnp.float32),
        pltpu.VMEM((_P,), jnp.float32),
        pltpu.VMEM((_P,), jnp.float32),
        pltpu.VMEM((_P,), jnp.int32),
        pltpu.VMEM((_P,), jnp.int32),
        pltpu.VMEM((_P,), jnp.float32),
        pltpu.VMEM((_P,), jnp.float32),
        pltpu.VMEM((_P,), jnp.float32),
        pltpu.VMEM((_P,), jnp.float32),
        pltpu.VMEM((_P,), jnp.float32),
        pltpu.VMEM((_P,), jnp.float32),
        pltpu.VMEM((_SEG_PAD,), jnp.float32),
        pltpu.VMEM((_NS, _SEG_W), jnp.float32),
        pltpu.VMEM((48,), jnp.float32),
        pltpu.VMEM_SHARED((_NS, _SEG_PAD), jnp.float32),
        pltpu.SemaphoreType.DMA,
        pltpu.SemaphoreType.DMA,
        pltpu.SemaphoreType.DMA,
        pltpu.SemaphoreType.DMA,
    ],
    compiler_params=pltpu.CompilerParams(needs_layout_passes=False),
  )


def kernel(positions, segment_ids, minimum):
  xs = positions[:, 0]
  ys = positions[:, 1]
  zs = positions[:, 2]
  min_bcast = jnp.repeat(minimum.astype(jnp.float32), 16)
  eng2, gx, gy, gz = _sc_call(xs, ys, zs, segment_ids, min_bcast)
  energies = (eng2[0] + eng2[1])[:_NUM_SEG]
  neg_grad = jnp.stack([gx, gy, gz], axis=1)
  stress = jnp.zeros((_NUM_SEG, 6), positions.dtype)
  return (energies, neg_grad, stress)
